# Initial kernel scaffold; baseline (speedup 1.0000x reference)
import jax
import jax.numpy as jnp
from jax.experimental import pallas as pl


def kernel(x_pos, edge_index_pos, edge_attr_pos, batch_pos, x_neg, edge_index_neg, edge_attr_neg, batch_neg, W1, b1, W2, b2, P1W, P1b, P2W, P2b):
    raise NotImplementedError("write your pallas kernel here")


# speedup vs baseline: 12.5527x; 12.5527x over previous
import functools

import jax
import jax.numpy as jnp
from jax import lax
from jax.experimental import pallas as pl
from jax.experimental.pallas import tpu as pltpu
from jax.experimental.pallas import tpu_sc as plsc

_N = 10000
_E = 320000
_D = 128
_B = 128
_NPAD = 10240

_NC = 2
_NS = 16
_RPT = _NPAD // _NS

_AK = 125
_ACH = _E // _NS // _AK
_BK = 125
_BCH = _E // (_NC * _NS) // _BK

_RB = 512
_G = _NPAD // _RB


def _mesh():
    return plsc.VectorSubcoreMesh(
        core_axis_name="c", subcore_axis_name="s", num_cores=_NC, num_subcores=_NS
    )


def _sc_degree(dst_r, w_r):
    @functools.partial(
        pl.kernel,
        out_type=jax.ShapeDtypeStruct((_NC, _NPAD), jnp.float32),
        mesh=_mesh(),
        compiler_params=pltpu.CompilerParams(use_tc_tiling_on_sc=False),
        scratch_types=[
            pltpu.VMEM((_ACH, _AK), jnp.int32),
            pltpu.VMEM((_ACH, _AK), jnp.float32),
            pltpu.VMEM((_RPT,), jnp.float32),
            pltpu.VMEM_SHARED((_NPAD,), jnp.float32),
        ],
    )
    def body(dst_hbm, w_hbm, out_hbm, dst_v, w_v, zer_v, acc):
        c = lax.axis_index("c")
        s = lax.axis_index("s")

        def zfill(i, _):
            zer_v[pl.ds(i * 16, 16)] = jnp.zeros((16,), jnp.float32)
            return 0

        lax.fori_loop(0, _RPT // 16, zfill, 0)
        pltpu.sync_copy(zer_v, acc.at[pl.ds(s * _RPT, _RPT)])
        plsc.subcore_barrier()

        pltpu.sync_copy(dst_hbm.at[c, s], dst_v)
        pltpu.sync_copy(w_hbm.at[c, s], w_v)

        def chunk(j, _):
            pltpu.sync_copy(w_v.at[j], acc.at[dst_v.at[j]], add=True)
            return 0

        lax.fori_loop(0, _ACH, chunk, 0)
        plsc.subcore_barrier()
        pltpu.sync_copy(
            acc.at[pl.ds(s * _RPT, _RPT)], out_hbm.at[c, pl.ds(s * _RPT, _RPT)]
        )

    return body(dst_r, w_r)


def _sc_conv(src_r, dst_r, w_r, xw):
    _EB = _E // (_NC * _NS)

    @functools.partial(
        pl.kernel,
        out_type=jax.ShapeDtypeStruct((_NC, _NPAD, _D), jnp.float32),
        mesh=_mesh(),
        compiler_params=pltpu.CompilerParams(use_tc_tiling_on_sc=False),
        scratch_types=[
            pltpu.VMEM((_BCH, _BK), jnp.int32),
            pltpu.VMEM((_BCH, _BK), jnp.int32),
            pltpu.VMEM((_EB + 16,), jnp.float32),
            pltpu.VMEM((_BK, _D), jnp.float32),
            pltpu.VMEM_SHARED((_NPAD, _D), jnp.float32),
        ],
    )
    def body(src_hbm, dst_hbm, w_hbm, xw_hbm, out_hbm, src_v, dst_v, w_v, rows, acc):
        c = lax.axis_index("c")
        s = lax.axis_index("s")

        def zfill(i, _):
            rows[i // 8, pl.ds((i % 8) * 16, 16)] = jnp.zeros((16,), jnp.float32)
            return 0

        lax.fori_loop(0, _BK * 8, zfill, 0)

        def zcopy(j, _):
            pltpu.sync_copy(rows.at[pl.ds(0, 80)], acc.at[pl.ds(s * _RPT + j * 80, 80)])
            return 0

        lax.fori_loop(0, _RPT // 80, zcopy, 0)
        plsc.subcore_barrier()

        pltpu.sync_copy(src_hbm.at[c, s], src_v)
        pltpu.sync_copy(dst_hbm.at[c, s], dst_v)
        pltpu.sync_copy(w_hbm.at[c, s], w_v)

        def chunk(j, _):
            pltpu.sync_copy(xw_hbm.at[src_v.at[j]], rows)

            def row(r, _):
                sn = w_v[pl.ds(j * _BK + r, 16)][0]
                for k in range(_D // 16):
                    rows[r, pl.ds(k * 16, 16)] = rows[r, pl.ds(k * 16, 16)] * sn
                return 0

            lax.fori_loop(0, _BK, row, 0)
            pltpu.sync_copy(rows, acc.at[dst_v.at[j]], add=True)
            return 0

        lax.fori_loop(0, _BCH, chunk, 0)
        plsc.subcore_barrier()

        def ocopy(j, _):
            pltpu.sync_copy(
                acc.at[pl.ds(s * _RPT + j * 80, 80)],
                out_hbm.at[c, pl.ds(s * _RPT + j * 80, 80)],
            )
            return 0

        lax.fori_loop(0, _RPT // 80, ocopy, 0)

    return body(src_r, dst_r, w_r, xw)


def _dinv_block(p_blk):
    deg = 1.0 + p_blk
    return jnp.where(deg > 0, lax.rsqrt(jnp.maximum(deg, 1e-12)), 0.0)


def _tc_xw(x, W, p):
    def body(x_ref, w_ref, p_ref, o_ref):
        dinv = _dinv_block(p_ref[...])
        xw = jnp.dot(x_ref[...], w_ref[...], preferred_element_type=jnp.float32)
        o_ref[...] = xw * dinv

    return pl.pallas_call(
        body,
        grid=(_G,),
        in_specs=[
            pl.BlockSpec((_RB, _D), lambda i: (i, 0)),
            pl.BlockSpec((_D, _D), lambda i: (0, 0)),
            pl.BlockSpec((_RB, 1), lambda i: (i, 0)),
        ],
        out_specs=pl.BlockSpec((_RB, _D), lambda i: (i, 0)),
        out_shape=jax.ShapeDtypeStruct((_NPAD, _D), jnp.float32),
    )(x, W, p)


def _tc_mid(S, xwp, p, b, W2):
    def body(s0_ref, s1_ref, xw_ref, p_ref, b_ref, w_ref, o_ref):
        dinv = _dinv_block(p_ref[...])
        h = dinv * (s0_ref[0] + s1_ref[0] + xw_ref[...]) + b_ref[...]
        h = jnp.maximum(h, 0.0)
        o_ref[...] = (
            jnp.dot(h, w_ref[...], preferred_element_type=jnp.float32) * dinv
        )

    return pl.pallas_call(
        body,
        grid=(_G,),
        in_specs=[
            pl.BlockSpec((1, _RB, _D), lambda i: (0, i, 0)),
            pl.BlockSpec((1, _RB, _D), lambda i: (1, i, 0)),
            pl.BlockSpec((_RB, _D), lambda i: (i, 0)),
            pl.BlockSpec((_RB, 1), lambda i: (i, 0)),
            pl.BlockSpec((1, _D), lambda i: (0, 0)),
            pl.BlockSpec((_D, _D), lambda i: (0, 0)),
        ],
        out_specs=pl.BlockSpec((_RB, _D), lambda i: (i, 0)),
        out_shape=jax.ShapeDtypeStruct((_NPAD, _D), jnp.float32),
    )(S, S, xwp, p, b, W2)


def _tc_pool(S, xwp, p, b, batch3):
    def body(s0_ref, s1_ref, xw_ref, p_ref, b_ref, bt_ref, sum_ref, cnt_ref):
        i = pl.program_id(0)

        @pl.when(i == 0)
        def _():
            sum_ref[...] = jnp.zeros_like(sum_ref)
            cnt_ref[...] = jnp.zeros_like(cnt_ref)

        dinv = _dinv_block(p_ref[...])
        z = dinv * (s0_ref[0] + s1_ref[0] + xw_ref[...]) + b_ref[...]
        seg = bt_ref[0]
        ids = lax.broadcasted_iota(jnp.int32, (_B, _RB), 0)
        onehot = (seg == ids).astype(jnp.float32)
        sum_ref[...] += jnp.dot(onehot, z, preferred_element_type=jnp.float32)
        cnt_ref[...] += jnp.broadcast_to(
            jnp.sum(onehot, axis=1, keepdims=True), (_B, _D)
        )

    return pl.pallas_call(
        body,
        grid=(_G,),
        in_specs=[
            pl.BlockSpec((1, _RB, _D), lambda i: (0, i, 0)),
            pl.BlockSpec((1, _RB, _D), lambda i: (1, i, 0)),
            pl.BlockSpec((_RB, _D), lambda i: (i, 0)),
            pl.BlockSpec((_RB, 1), lambda i: (i, 0)),
            pl.BlockSpec((1, _D), lambda i: (0, 0)),
            pl.BlockSpec((1, 1, _RB), lambda i: (i, 0, 0)),
        ],
        out_specs=[
            pl.BlockSpec((_B, _D), lambda i: (0, 0)),
            pl.BlockSpec((_B, _D), lambda i: (0, 0)),
        ],
        out_shape=[
            jax.ShapeDtypeStruct((_B, _D), jnp.float32),
            jax.ShapeDtypeStruct((_B, _D), jnp.float32),
        ],
    )(S, S, xwp, p, b, batch3)


def _tc_head(sp, cp, sn, cn, p1a, p1b_w, p1bias, p2w, p2bias):
    def body(sp_ref, cp_ref, sn_ref, cn_ref, a_ref, bw_ref, pb_ref, w2_ref, b2_ref, o_ref):
        mp = sp_ref[...] / jnp.maximum(cp_ref[...], 1.0)
        mn = sn_ref[...] / jnp.maximum(cn_ref[...], 1.0)
        h = (
            jnp.dot(mp, a_ref[...], preferred_element_type=jnp.float32)
            + jnp.dot(mn, bw_ref[...], preferred_element_type=jnp.float32)
            + pb_ref[...]
        )
        h = jnp.maximum(h, 0.0)
        o_ref[...] = (
            jnp.dot(h, w2_ref[...], preferred_element_type=jnp.float32) + b2_ref[...]
        )

    return pl.pallas_call(
        body,
        out_shape=jax.ShapeDtypeStruct((_B, _D), jnp.float32),
    )(sp, cp, sn, cn, p1a, p1b_w, p1bias, p2w, p2bias)


@jax.jit
def kernel(
    x_pos, edge_index_pos, edge_attr_pos, batch_pos,
    x_neg, edge_index_neg, edge_attr_neg, batch_neg,
    W1, b1, W2, b2, P1W, P1b, P2W, P2b,
):
    b1r = b1.reshape(1, _D)
    b2r = b2.reshape(1, _D)
    p1a = P1W[:_D]
    p1b_w = P1W[_D:]
    p1bias = P1b.reshape(1, _D)
    p2w = jnp.pad(P2W, ((0, 0), (0, _D - P2W.shape[1])))
    p2b = jnp.pad(P2b, (0, _D - P2b.shape[0])).reshape(1, _D)

    dstA = jnp.stack(
        [
            edge_index_pos[1].reshape(_NS, _ACH, _AK),
            edge_index_neg[1].reshape(_NS, _ACH, _AK),
        ]
    )
    wA = jnp.stack(
        [
            edge_attr_pos.reshape(_NS, _ACH, _AK),
            edge_attr_neg.reshape(_NS, _ACH, _AK),
        ]
    )
    degp = _sc_degree(dstA, wA)

    pooled = []
    for g, (x, ei, ew, bt) in enumerate(
        (
            (x_pos, edge_index_pos, edge_attr_pos, batch_pos),
            (x_neg, edge_index_neg, edge_attr_neg, batch_neg),
        )
    ):
        p = degp[g].reshape(_NPAD, 1)
        xpad = jnp.pad(x, ((0, _NPAD - _N), (0, 0)))
        src_r = ei[0].reshape(_NC, _NS, _BCH, _BK)
        dst_r = ei[1].reshape(_NC, _NS, _BCH, _BK)
        w_r = jnp.pad(
            ew.reshape(_NC, _NS, _BCH * _BK), ((0, 0), (0, 0), (0, 16))
        )
        bt3 = jnp.pad(bt, (0, _NPAD - _N), constant_values=_B).reshape(_G, 1, _RB)

        xw1 = _tc_xw(xpad, W1, p)
        S1 = _sc_conv(src_r, dst_r, w_r, xw1)
        xw2 = _tc_mid(S1, xw1, p, b1r, W2)
        S2 = _sc_conv(src_r, dst_r, w_r, xw2)
        sm, ct = _tc_pool(S2, xw2, p, b2r, bt3)
        pooled.append((sm, ct))

    full = _tc_head(
        pooled[0][0], pooled[0][1], pooled[1][0], pooled[1][1],
        p1a, p1b_w, p1bias, p2w, p2b,
    )
    return full[:, : P2W.shape[1]]


# speedup vs baseline: 17.9489x; 1.4299x over previous
import functools

import jax
import jax.numpy as jnp
from jax import lax
from jax.experimental import pallas as pl
from jax.experimental.pallas import tpu as pltpu
from jax.experimental.pallas import tpu_sc as plsc

_N = 10000
_E = 320000
_D = 128
_B = 128
_NPAD = 10240

_NC = 2
_NS = 16
_RPT = _NPAD // _NS

_AK = 125
_ACH = _E // _NS // _AK
_BK = 128
_BCH = 80
_EPAD = _NC * _NS * _BCH * _BK

_RB = 512
_G = _NPAD // _RB


def _mesh():
    return plsc.VectorSubcoreMesh(
        core_axis_name="c", subcore_axis_name="s", num_cores=_NC, num_subcores=_NS
    )


def _sc_degree(dst_r, w_r):
    @functools.partial(
        pl.kernel,
        out_type=jax.ShapeDtypeStruct((_NC, _NPAD), jnp.float32),
        mesh=_mesh(),
        compiler_params=pltpu.CompilerParams(use_tc_tiling_on_sc=False),
        scratch_types=[
            pltpu.VMEM((_ACH, _AK), jnp.int32),
            pltpu.VMEM((_ACH, _AK), jnp.float32),
            pltpu.VMEM((_RPT,), jnp.float32),
            pltpu.VMEM_SHARED((_NPAD,), jnp.float32),
        ],
    )
    def body(dst_hbm, w_hbm, out_hbm, dst_v, w_v, zer_v, acc):
        c = lax.axis_index("c")
        s = lax.axis_index("s")

        def zfill(i, _):
            zer_v[pl.ds(i * 16, 16)] = jnp.zeros((16,), jnp.float32)
            return 0

        lax.fori_loop(0, _RPT // 16, zfill, 0)
        pltpu.sync_copy(zer_v, acc.at[pl.ds(s * _RPT, _RPT)])
        plsc.subcore_barrier()

        pltpu.sync_copy(dst_hbm.at[c, s], dst_v)
        pltpu.sync_copy(w_hbm.at[c, s], w_v)

        def chunk(j, _):
            pltpu.sync_copy(w_v.at[j], acc.at[dst_v.at[j]], add=True)
            return 0

        lax.fori_loop(0, _ACH, chunk, 0)
        plsc.subcore_barrier()
        pltpu.sync_copy(
            acc.at[pl.ds(s * _RPT, _RPT)], out_hbm.at[c, pl.ds(s * _RPT, _RPT)]
        )

    return body(dst_r, w_r)


def _sc_conv(idx_r, w_r, xw):
    @functools.partial(
        pl.kernel,
        out_type=jax.ShapeDtypeStruct((_NC, _NPAD, _D), jnp.float32),
        mesh=_mesh(),
        compiler_params=pltpu.CompilerParams(use_tc_tiling_on_sc=False),
        scratch_types=[
            pltpu.VMEM((2, 2, _BK), jnp.int32),
            pltpu.VMEM((2, _BK + 16), jnp.float32),
            pltpu.VMEM((2, _BK, _D), jnp.float32),
            pltpu.VMEM_SHARED((_NPAD, _D), jnp.float32),
            pltpu.SemaphoreType.DMA,
            pltpu.SemaphoreType.DMA,
            pltpu.SemaphoreType.DMA,
            pltpu.SemaphoreType.DMA,
            pltpu.SemaphoreType.DMA,
            pltpu.SemaphoreType.DMA,
        ],
    )
    def body(idx_hbm, w_hbm, xw_hbm, out_hbm, idxv, wv, rows, acc,
             isem0, isem1, gsem0, gsem1, ssem0, ssem1):
        c = lax.axis_index("c")
        s = lax.axis_index("s")
        isem = (isem0, isem1)
        gsem = (gsem0, gsem1)
        ssem = (ssem0, ssem1)

        def zfill(i, _):
            rows[0, i // 8, pl.ds((i % 8) * 16, 16)] = jnp.zeros((16,), jnp.float32)
            return 0

        lax.fori_loop(0, _BK * 8, zfill, 0)

        def zcopy(j, _):
            pltpu.sync_copy(
                rows.at[0], acc.at[pl.ds(s * _RPT + j * _BK, _BK)]
            )
            return 0

        lax.fori_loop(0, _RPT // _BK, zcopy, 0)
        plsc.subcore_barrier()

        def compute(b):
            def row(r, _):
                sn = wv[b, pl.ds(r, 16)][0]
                for k in range(_D // 16):
                    rows[b, r, pl.ds(k * 16, 16)] = rows[b, r, pl.ds(k * 16, 16)] * sn
                return 0

            lax.fori_loop(0, _BK, row, 0, unroll=2)

        def load_idx(j, b):
            pltpu.async_copy(idx_hbm.at[c, s, j], idxv.at[b], isem[b])
            pltpu.async_copy(w_hbm.at[c, s, j], wv.at[b], isem[b])

        def wait_idx(b):
            pltpu.make_async_copy(idx_hbm.at[c, s, 0], idxv.at[b], isem[b]).wait()
            pltpu.make_async_copy(w_hbm.at[c, s, 0], wv.at[b], isem[b]).wait()

        def issue_gather(b):
            pltpu.async_copy(xw_hbm.at[idxv.at[b, 0]], rows.at[b], gsem[b])

        def wait_gather(b):
            pltpu.make_async_copy(xw_hbm.at[idxv.at[b, 0]], rows.at[b], gsem[b]).wait()

        def issue_scatter(b):
            pltpu.async_copy(rows.at[b], acc.at[idxv.at[b, 1]], ssem[b], add=True)

        def wait_scatter(b):
            pltpu.make_async_copy(rows.at[b], acc.at[idxv.at[b, 1]], ssem[b]).wait()

        load_idx(0, 0)
        wait_idx(0)
        issue_gather(0)

        def pair(jj, _):
            j0 = jj * 2
            @pl.when(jj > 0)
            def _():
                wait_scatter(1)

            load_idx(j0 + 1, 1)
            wait_idx(1)
            issue_gather(1)
            wait_gather(0)
            compute(0)
            issue_scatter(0)
            wait_gather(1)
            compute(1)
            issue_scatter(1)

            @pl.when(jj < _BCH // 2 - 1)
            def _():
                wait_scatter(0)
                load_idx(j0 + 2, 0)
                wait_idx(0)
                issue_gather(0)

            return 0

        lax.fori_loop(0, _BCH // 2, pair, 0)
        wait_scatter(0)
        wait_scatter(1)
        plsc.subcore_barrier()

        def ocopy(j, _):
            pltpu.sync_copy(
                acc.at[pl.ds(s * _RPT + j * _BK, _BK)],
                out_hbm.at[c, pl.ds(s * _RPT + j * _BK, _BK)],
            )
            return 0

        lax.fori_loop(0, _RPT // _BK, ocopy, 0)

    return body(idx_r, w_r, xw)


def _dinv_block(p_blk):
    deg = 1.0 + p_blk
    return jnp.where(deg > 0, lax.rsqrt(jnp.maximum(deg, 1e-12)), 0.0)


def _tc_xw(x, W, p):
    def body(x_ref, w_ref, p_ref, o_ref):
        dinv = _dinv_block(p_ref[...])
        xw = jnp.dot(x_ref[...], w_ref[...], preferred_element_type=jnp.float32)
        o_ref[...] = xw * dinv

    return pl.pallas_call(
        body,
        grid=(_G,),
        in_specs=[
            pl.BlockSpec((_RB, _D), lambda i: (i, 0)),
            pl.BlockSpec((_D, _D), lambda i: (0, 0)),
            pl.BlockSpec((_RB, 1), lambda i: (i, 0)),
        ],
        out_specs=pl.BlockSpec((_RB, _D), lambda i: (i, 0)),
        out_shape=jax.ShapeDtypeStruct((_NPAD, _D), jnp.float32),
    )(x, W, p)


def _tc_mid(S, xwp, p, b, W2):
    def body(s0_ref, s1_ref, xw_ref, p_ref, b_ref, w_ref, o_ref):
        dinv = _dinv_block(p_ref[...])
        h = dinv * (s0_ref[0] + s1_ref[0] + xw_ref[...]) + b_ref[...]
        h = jnp.maximum(h, 0.0)
        o_ref[...] = (
            jnp.dot(h, w_ref[...], preferred_element_type=jnp.float32) * dinv
        )

    return pl.pallas_call(
        body,
        grid=(_G,),
        in_specs=[
            pl.BlockSpec((1, _RB, _D), lambda i: (0, i, 0)),
            pl.BlockSpec((1, _RB, _D), lambda i: (1, i, 0)),
            pl.BlockSpec((_RB, _D), lambda i: (i, 0)),
            pl.BlockSpec((_RB, 1), lambda i: (i, 0)),
            pl.BlockSpec((1, _D), lambda i: (0, 0)),
            pl.BlockSpec((_D, _D), lambda i: (0, 0)),
        ],
        out_specs=pl.BlockSpec((_RB, _D), lambda i: (i, 0)),
        out_shape=jax.ShapeDtypeStruct((_NPAD, _D), jnp.float32),
    )(S, S, xwp, p, b, W2)


def _tc_pool(S, xwp, p, b, batch3):
    def body(s0_ref, s1_ref, xw_ref, p_ref, b_ref, bt_ref, sum_ref, cnt_ref):
        i = pl.program_id(0)

        @pl.when(i == 0)
        def _():
            sum_ref[...] = jnp.zeros_like(sum_ref)
            cnt_ref[...] = jnp.zeros_like(cnt_ref)

        dinv = _dinv_block(p_ref[...])
        z = dinv * (s0_ref[0] + s1_ref[0] + xw_ref[...]) + b_ref[...]
        seg = bt_ref[0]
        ids = lax.broadcasted_iota(jnp.int32, (_B, _RB), 0)
        onehot = (seg == ids).astype(jnp.float32)
        sum_ref[...] += jnp.dot(onehot, z, preferred_element_type=jnp.float32)
        cnt_ref[...] += jnp.broadcast_to(
            jnp.sum(onehot, axis=1, keepdims=True), (_B, _D)
        )

    return pl.pallas_call(
        body,
        grid=(_G,),
        in_specs=[
            pl.BlockSpec((1, _RB, _D), lambda i: (0, i, 0)),
            pl.BlockSpec((1, _RB, _D), lambda i: (1, i, 0)),
            pl.BlockSpec((_RB, _D), lambda i: (i, 0)),
            pl.BlockSpec((_RB, 1), lambda i: (i, 0)),
            pl.BlockSpec((1, _D), lambda i: (0, 0)),
            pl.BlockSpec((1, 1, _RB), lambda i: (i, 0, 0)),
        ],
        out_specs=[
            pl.BlockSpec((_B, _D), lambda i: (0, 0)),
            pl.BlockSpec((_B, _D), lambda i: (0, 0)),
        ],
        out_shape=[
            jax.ShapeDtypeStruct((_B, _D), jnp.float32),
            jax.ShapeDtypeStruct((_B, _D), jnp.float32),
        ],
    )(S, S, xwp, p, b, batch3)


def _tc_head(sp, cp, sn, cn, p1a, p1b_w, p1bias, p2w, p2bias):
    def body(sp_ref, cp_ref, sn_ref, cn_ref, a_ref, bw_ref, pb_ref, w2_ref, b2_ref, o_ref):
        mp = sp_ref[...] / jnp.maximum(cp_ref[...], 1.0)
        mn = sn_ref[...] / jnp.maximum(cn_ref[...], 1.0)
        h = (
            jnp.dot(mp, a_ref[...], preferred_element_type=jnp.float32)
            + jnp.dot(mn, bw_ref[...], preferred_element_type=jnp.float32)
            + pb_ref[...]
        )
        h = jnp.maximum(h, 0.0)
        o_ref[...] = (
            jnp.dot(h, w2_ref[...], preferred_element_type=jnp.float32) + b2_ref[...]
        )

    return pl.pallas_call(
        body,
        out_shape=jax.ShapeDtypeStruct((_B, _D), jnp.float32),
    )(sp, cp, sn, cn, p1a, p1b_w, p1bias, p2w, p2bias)


@jax.jit
def kernel(
    x_pos, edge_index_pos, edge_attr_pos, batch_pos,
    x_neg, edge_index_neg, edge_attr_neg, batch_neg,
    W1, b1, W2, b2, P1W, P1b, P2W, P2b,
):
    b1r = b1.reshape(1, _D)
    b2r = b2.reshape(1, _D)
    p1a = P1W[:_D]
    p1b_w = P1W[_D:]
    p1bias = P1b.reshape(1, _D)
    p2w = jnp.pad(P2W, ((0, 0), (0, _D - P2W.shape[1])))
    p2b = jnp.pad(P2b, (0, _D - P2b.shape[0])).reshape(1, _D)

    dstA = jnp.stack(
        [
            edge_index_pos[1].reshape(_NS, _ACH, _AK),
            edge_index_neg[1].reshape(_NS, _ACH, _AK),
        ]
    )
    wA = jnp.stack(
        [
            edge_attr_pos.reshape(_NS, _ACH, _AK),
            edge_attr_neg.reshape(_NS, _ACH, _AK),
        ]
    )
    degp = _sc_degree(dstA, wA)

    pooled = []
    for g, (x, ei, ew, bt) in enumerate(
        (
            (x_pos, edge_index_pos, edge_attr_pos, batch_pos),
            (x_neg, edge_index_neg, edge_attr_neg, batch_neg),
        )
    ):
        p = degp[g].reshape(_NPAD, 1)
        xpad = jnp.pad(x, ((0, _NPAD - _N), (0, 0)))
        npad_e = _EPAD - _E
        fill = (jnp.arange(npad_e, dtype=jnp.int32) * 131) % _N
        srcp = jnp.concatenate([ei[0], fill]).reshape(_NC, _NS, _BCH, _BK)
        dstp = jnp.concatenate([ei[1], fill]).reshape(_NC, _NS, _BCH, _BK)
        idx_r = jnp.stack([srcp, dstp], axis=3)
        w_r = jnp.pad(
            jnp.concatenate([ew, jnp.zeros((npad_e,), jnp.float32)]).reshape(
                _NC, _NS, _BCH, _BK
            ),
            ((0, 0), (0, 0), (0, 0), (0, 16)),
        )
        bt3 = jnp.pad(bt, (0, _NPAD - _N), constant_values=_B).reshape(_G, 1, _RB)

        xw1 = _tc_xw(xpad, W1, p)
        S1 = _sc_conv(idx_r, w_r, xw1)
        xw2 = _tc_mid(S1, xw1, p, b1r, W2)
        S2 = _sc_conv(idx_r, w_r, xw2)
        sm, ct = _tc_pool(S2, xw2, p, b2r, bt3)
        pooled.append((sm, ct))

    full = _tc_head(
        pooled[0][0], pooled[0][1], pooled[1][0], pooled[1][1],
        p1a, p1b_w, p1bias, p2w, p2b,
    )
    return full[:, : P2W.shape[1]]


# speedup vs baseline: 19.4730x; 1.0849x over previous
import functools

import jax
import jax.numpy as jnp
from jax import lax
from jax.experimental import pallas as pl
from jax.experimental.pallas import tpu as pltpu
from jax.experimental.pallas import tpu_sc as plsc

_N = 10000
_E = 320000
_D = 128
_B = 128
_NPAD = 10240

_NC = 2
_NS = 16
_RPT = _NPAD // _NS

_AK = 125
_ACH = _E // _NS // _AK
_BK = 128
_BCH = 80
_EPAD = _NC * _NS * _BCH * _BK

_RB = 512
_G = _NPAD // _RB


def _mesh():
    return plsc.VectorSubcoreMesh(
        core_axis_name="c", subcore_axis_name="s", num_cores=_NC, num_subcores=_NS
    )


def _sc_degree(dst_r, w_r):
    @functools.partial(
        pl.kernel,
        out_type=jax.ShapeDtypeStruct((_NC, _NPAD), jnp.float32),
        mesh=_mesh(),
        compiler_params=pltpu.CompilerParams(use_tc_tiling_on_sc=False),
        scratch_types=[
            pltpu.VMEM((_ACH, _AK), jnp.int32),
            pltpu.VMEM((_ACH, _AK), jnp.float32),
            pltpu.VMEM((_RPT,), jnp.float32),
            pltpu.VMEM_SHARED((_NPAD,), jnp.float32),
        ],
    )
    def body(dst_hbm, w_hbm, out_hbm, dst_v, w_v, zer_v, acc):
        c = lax.axis_index("c")
        s = lax.axis_index("s")

        def zfill(i, _):
            zer_v[pl.ds(i * 16, 16)] = jnp.zeros((16,), jnp.float32)
            return 0

        lax.fori_loop(0, _RPT // 16, zfill, 0)
        pltpu.sync_copy(zer_v, acc.at[pl.ds(s * _RPT, _RPT)])
        plsc.subcore_barrier()

        pltpu.sync_copy(dst_hbm.at[c, s], dst_v)
        pltpu.sync_copy(w_hbm.at[c, s], w_v)

        def chunk(j, _):
            pltpu.sync_copy(w_v.at[j], acc.at[dst_v.at[j]], add=True)
            return 0

        lax.fori_loop(0, _ACH, chunk, 0)
        plsc.subcore_barrier()
        pltpu.sync_copy(
            acc.at[pl.ds(s * _RPT, _RPT)], out_hbm.at[c, pl.ds(s * _RPT, _RPT)]
        )

    return body(dst_r, w_r)


def _sc_conv(idx_r, w_r, xw):
    @functools.partial(
        pl.kernel,
        out_type=jax.ShapeDtypeStruct((_NC, _NPAD, _D), jnp.float32),
        mesh=_mesh(),
        compiler_params=pltpu.CompilerParams(use_tc_tiling_on_sc=False),
        scratch_types=[
            pltpu.VMEM((2, 2, 2, _BK), jnp.int32),
            pltpu.VMEM((2, 2, _BK + 16), jnp.float32),
            pltpu.VMEM((2, _BK, _D), jnp.float32),
            pltpu.VMEM_SHARED((_NPAD, _D), jnp.float32),
            pltpu.SemaphoreType.DMA,
            pltpu.SemaphoreType.DMA,
            pltpu.SemaphoreType.DMA,
            pltpu.SemaphoreType.DMA,
            pltpu.SemaphoreType.DMA,
        ],
    )
    def body(idx_hbm, w_hbm, xw_hbm, out_hbm, idxv, wv, rows, acc,
             isem, gsem0, gsem1, ssem0, ssem1):
        c = lax.axis_index("c")
        s = lax.axis_index("s")
        gsem = (gsem0, gsem1)
        ssem = (ssem0, ssem1)
        _HP = _BCH // 2

        def zfill(i, _):
            rows[0, i // 8, pl.ds((i % 8) * 16, 16)] = jnp.zeros((16,), jnp.float32)
            return 0

        lax.fori_loop(0, _BK * 8, zfill, 0)

        def zcopy(j, _):
            pltpu.sync_copy(
                rows.at[0], acc.at[pl.ds(s * _RPT + j * _BK, _BK)]
            )
            return 0

        lax.fori_loop(0, _RPT // _BK, zcopy, 0)
        plsc.subcore_barrier()

        def compute(sl, b):
            def row(r, _):
                sn = wv[sl, b, pl.ds(r, 16)][0]
                for k in range(_D // 16):
                    rows[b, r, pl.ds(k * 16, 16)] = rows[b, r, pl.ds(k * 16, 16)] * sn
                return 0

            lax.fori_loop(0, _BK, row, 0, unroll=4)

        def load_idx_pair(j0, sl):
            pltpu.async_copy(idx_hbm.at[c, s, j0], idxv.at[sl, 0], isem)
            pltpu.async_copy(w_hbm.at[c, s, j0], wv.at[sl, 0], isem)
            pltpu.async_copy(idx_hbm.at[c, s, j0 + 1], idxv.at[sl, 1], isem)
            pltpu.async_copy(w_hbm.at[c, s, j0 + 1], wv.at[sl, 1], isem)

        def wait_idx_pair():
            for b in range(2):
                pltpu.make_async_copy(idx_hbm.at[c, s, 0], idxv.at[0, b], isem).wait()
                pltpu.make_async_copy(w_hbm.at[c, s, 0], wv.at[0, b], isem).wait()

        def issue_gather(sl, b):
            pltpu.async_copy(xw_hbm.at[idxv.at[sl, b, 0]], rows.at[b], gsem[b])

        def wait_gather(b):
            pltpu.make_async_copy(
                xw_hbm.at[idxv.at[0, b, 0]], rows.at[b], gsem[b]
            ).wait()

        def issue_scatter(sl, b):
            pltpu.async_copy(rows.at[b], acc.at[idxv.at[sl, b, 1]], ssem[b], add=True)

        def wait_scatter(b):
            pltpu.make_async_copy(rows.at[b], acc.at[idxv.at[0, b, 1]], ssem[b]).wait()

        load_idx_pair(0, 0)
        wait_idx_pair()
        issue_gather(0, 0)

        def sub_pair(jj, sl, sln):
            @pl.when(jj > 0)
            def _():
                wait_scatter(1)

            issue_gather(sl, 1)

            @pl.when(jj < _HP - 1)
            def _():
                load_idx_pair(jj * 2 + 2, sln)

            wait_gather(0)
            compute(sl, 0)
            issue_scatter(sl, 0)
            wait_gather(1)
            compute(sl, 1)
            issue_scatter(sl, 1)

            @pl.when(jj < _HP - 1)
            def _():
                wait_scatter(0)
                wait_idx_pair()
                issue_gather(sln, 0)

        def super_pair(kk, _):
            sub_pair(kk * 2, 0, 1)
            sub_pair(kk * 2 + 1, 1, 0)
            return 0

        lax.fori_loop(0, _HP // 2, super_pair, 0)
        wait_scatter(0)
        wait_scatter(1)
        plsc.subcore_barrier()

        def ocopy(j, _):
            pltpu.sync_copy(
                acc.at[pl.ds(s * _RPT + j * _BK, _BK)],
                out_hbm.at[c, pl.ds(s * _RPT + j * _BK, _BK)],
            )
            return 0

        lax.fori_loop(0, _RPT // _BK, ocopy, 0)

    return body(idx_r, w_r, xw)


def _dinv_block(p_blk):
    deg = 1.0 + p_blk
    return jnp.where(deg > 0, lax.rsqrt(jnp.maximum(deg, 1e-12)), 0.0)


def _tc_xw(x, W, p):
    def body(x_ref, w_ref, p_ref, o_ref):
        dinv = _dinv_block(p_ref[...])
        xw = jnp.dot(x_ref[...], w_ref[...], preferred_element_type=jnp.float32)
        o_ref[...] = xw * dinv

    return pl.pallas_call(
        body,
        grid=(_G,),
        in_specs=[
            pl.BlockSpec((_RB, _D), lambda i: (i, 0)),
            pl.BlockSpec((_D, _D), lambda i: (0, 0)),
            pl.BlockSpec((_RB, 1), lambda i: (i, 0)),
        ],
        out_specs=pl.BlockSpec((_RB, _D), lambda i: (i, 0)),
        out_shape=jax.ShapeDtypeStruct((_NPAD, _D), jnp.float32),
    )(x, W, p)


def _tc_mid(S, xwp, p, b, W2):
    def body(s0_ref, s1_ref, xw_ref, p_ref, b_ref, w_ref, o_ref):
        dinv = _dinv_block(p_ref[...])
        h = dinv * (s0_ref[0] + s1_ref[0] + xw_ref[...]) + b_ref[...]
        h = jnp.maximum(h, 0.0)
        o_ref[...] = (
            jnp.dot(h, w_ref[...], preferred_element_type=jnp.float32) * dinv
        )

    return pl.pallas_call(
        body,
        grid=(_G,),
        in_specs=[
            pl.BlockSpec((1, _RB, _D), lambda i: (0, i, 0)),
            pl.BlockSpec((1, _RB, _D), lambda i: (1, i, 0)),
            pl.BlockSpec((_RB, _D), lambda i: (i, 0)),
            pl.BlockSpec((_RB, 1), lambda i: (i, 0)),
            pl.BlockSpec((1, _D), lambda i: (0, 0)),
            pl.BlockSpec((_D, _D), lambda i: (0, 0)),
        ],
        out_specs=pl.BlockSpec((_RB, _D), lambda i: (i, 0)),
        out_shape=jax.ShapeDtypeStruct((_NPAD, _D), jnp.float32),
    )(S, S, xwp, p, b, W2)


def _tc_pool(S, xwp, p, b, batch3):
    def body(s0_ref, s1_ref, xw_ref, p_ref, b_ref, bt_ref, sum_ref, cnt_ref):
        i = pl.program_id(0)

        @pl.when(i == 0)
        def _():
            sum_ref[...] = jnp.zeros_like(sum_ref)
            cnt_ref[...] = jnp.zeros_like(cnt_ref)

        dinv = _dinv_block(p_ref[...])
        z = dinv * (s0_ref[0] + s1_ref[0] + xw_ref[...]) + b_ref[...]
        seg = bt_ref[0]
        ids = lax.broadcasted_iota(jnp.int32, (_B, _RB), 0)
        onehot = (seg == ids).astype(jnp.float32)
        sum_ref[...] += jnp.dot(onehot, z, preferred_element_type=jnp.float32)
        cnt_ref[...] += jnp.broadcast_to(
            jnp.sum(onehot, axis=1, keepdims=True), (_B, _D)
        )

    return pl.pallas_call(
        body,
        grid=(_G,),
        in_specs=[
            pl.BlockSpec((1, _RB, _D), lambda i: (0, i, 0)),
            pl.BlockSpec((1, _RB, _D), lambda i: (1, i, 0)),
            pl.BlockSpec((_RB, _D), lambda i: (i, 0)),
            pl.BlockSpec((_RB, 1), lambda i: (i, 0)),
            pl.BlockSpec((1, _D), lambda i: (0, 0)),
            pl.BlockSpec((1, 1, _RB), lambda i: (i, 0, 0)),
        ],
        out_specs=[
            pl.BlockSpec((_B, _D), lambda i: (0, 0)),
            pl.BlockSpec((_B, _D), lambda i: (0, 0)),
        ],
        out_shape=[
            jax.ShapeDtypeStruct((_B, _D), jnp.float32),
            jax.ShapeDtypeStruct((_B, _D), jnp.float32),
        ],
    )(S, S, xwp, p, b, batch3)


def _tc_head(sp, cp, sn, cn, p1a, p1b_w, p1bias, p2w, p2bias):
    def body(sp_ref, cp_ref, sn_ref, cn_ref, a_ref, bw_ref, pb_ref, w2_ref, b2_ref, o_ref):
        mp = sp_ref[...] / jnp.maximum(cp_ref[...], 1.0)
        mn = sn_ref[...] / jnp.maximum(cn_ref[...], 1.0)
        h = (
            jnp.dot(mp, a_ref[...], preferred_element_type=jnp.float32)
            + jnp.dot(mn, bw_ref[...], preferred_element_type=jnp.float32)
            + pb_ref[...]
        )
        h = jnp.maximum(h, 0.0)
        o_ref[...] = (
            jnp.dot(h, w2_ref[...], preferred_element_type=jnp.float32) + b2_ref[...]
        )

    return pl.pallas_call(
        body,
        out_shape=jax.ShapeDtypeStruct((_B, _D), jnp.float32),
    )(sp, cp, sn, cn, p1a, p1b_w, p1bias, p2w, p2bias)


@jax.jit
def kernel(
    x_pos, edge_index_pos, edge_attr_pos, batch_pos,
    x_neg, edge_index_neg, edge_attr_neg, batch_neg,
    W1, b1, W2, b2, P1W, P1b, P2W, P2b,
):
    b1r = b1.reshape(1, _D)
    b2r = b2.reshape(1, _D)
    p1a = P1W[:_D]
    p1b_w = P1W[_D:]
    p1bias = P1b.reshape(1, _D)
    p2w = jnp.pad(P2W, ((0, 0), (0, _D - P2W.shape[1])))
    p2b = jnp.pad(P2b, (0, _D - P2b.shape[0])).reshape(1, _D)

    dstA = jnp.stack(
        [
            edge_index_pos[1].reshape(_NS, _ACH, _AK),
            edge_index_neg[1].reshape(_NS, _ACH, _AK),
        ]
    )
    wA = jnp.stack(
        [
            edge_attr_pos.reshape(_NS, _ACH, _AK),
            edge_attr_neg.reshape(_NS, _ACH, _AK),
        ]
    )
    degp = _sc_degree(dstA, wA)

    pooled = []
    for g, (x, ei, ew, bt) in enumerate(
        (
            (x_pos, edge_index_pos, edge_attr_pos, batch_pos),
            (x_neg, edge_index_neg, edge_attr_neg, batch_neg),
        )
    ):
        p = degp[g].reshape(_NPAD, 1)
        xpad = jnp.pad(x, ((0, _NPAD - _N), (0, 0)))
        npad_e = _EPAD - _E
        fill = (jnp.arange(npad_e, dtype=jnp.int32) * 131) % _N
        srcp = jnp.concatenate([ei[0], fill]).reshape(_NC, _NS, _BCH, _BK)
        dstp = jnp.concatenate([ei[1], fill]).reshape(_NC, _NS, _BCH, _BK)
        idx_r = jnp.stack([srcp, dstp], axis=3)
        w_r = jnp.pad(
            jnp.concatenate([ew, jnp.zeros((npad_e,), jnp.float32)]).reshape(
                _NC, _NS, _BCH, _BK
            ),
            ((0, 0), (0, 0), (0, 0), (0, 16)),
        )
        bt3 = jnp.pad(bt, (0, _NPAD - _N), constant_values=_B).reshape(_G, 1, _RB)

        xw1 = _tc_xw(xpad, W1, p)
        S1 = _sc_conv(idx_r, w_r, xw1)
        xw2 = _tc_mid(S1, xw1, p, b1r, W2)
        S2 = _sc_conv(idx_r, w_r, xw2)
        sm, ct = _tc_pool(S2, xw2, p, b2r, bt3)
        pooled.append((sm, ct))

    full = _tc_head(
        pooled[0][0], pooled[0][1], pooled[1][0], pooled[1][1],
        p1a, p1b_w, p1bias, p2w, p2b,
    )
    return full[:, : P2W.shape[1]]
